# baseline (speedup 1.0000x reference)
import functools

import jax
import jax.numpy as jnp
from jax.experimental import pallas as pl
from jax.experimental.pallas import tpu as pltpu
from jax.experimental.pallas import tpu_sc as plsc

_F32 = jnp.float32


def _gather_sc(We, idx_flat):
    N = idx_flat.shape[0]
    H = We.shape[1]
    W = 128
    mesh = plsc.VectorSubcoreMesh(core_axis_name="core", subcore_axis_name="subcore")
    idx2 = idx_flat.reshape(1, N)

    @functools.partial(
        pl.kernel,
        out_type=jax.ShapeDtypeStruct((N, H), _F32),
        mesh=mesh,
    )
    def gather_kernel(table_hbm, i_hbm, o_hbm):
        def body(i_vmem, o_vmem):
            pltpu.sync_copy(table_hbm.at[i_vmem.at[0]], o_vmem)

        pltpu.emit_pipeline(
            body,
            grid=(N // W,),
            in_specs=[pl.BlockSpec((1, W), lambda i: (0, i))],
            out_specs=[pl.BlockSpec((W, H), lambda i: (i, 0))],
            core_axis_name=("core", "subcore"),
            dimension_semantics=(pltpu.PARALLEL,),
        )(i_hbm, o_hbm)

    return gather_kernel(We, idx2)


def _gru_cat(x, h, W1, b, W2, bh, n):
    rz = jax.nn.sigmoid(
        jnp.dot(jnp.concatenate([x, h], axis=1).astype(jnp.bfloat16), W1,
                preferred_element_type=_F32) + b)
    r = rz[:, :n]
    z = rz[:, n:]
    hc = jnp.tanh(
        jnp.dot(jnp.concatenate([x, h * r], axis=1).astype(jnp.bfloat16), W2,
                preferred_element_type=_F32) + bh)
    return z * h + (1.0 - z) * hc


def _scans_body(T, B, H,
                x_ref, w1f, bf_, w2f, bhf_,
                w1b, bb_, w2b, bhb_,
                w1o, bo_, w2o, bho_,
                ctx_ref, hs_ref):
    h0 = jnp.zeros((B, H), _F32)

    _BF = jnp.bfloat16
    W1f, Bf, W2f, Bhf = w1f[:].astype(_BF), bf_[:], w2f[:].astype(_BF), bhf_[:]
    W1b, Bb, W2b, Bhb = w1b[:].astype(_BF), bb_[:], w2b[:].astype(_BF), bhb_[:]

    def step_bi(t, carry):
        hf, hb = carry
        xf = x_ref[pl.ds(t, 1)][0]
        xb = x_ref[pl.ds(T - 1 - t, 1)][0]
        hf = _gru_cat(xf, hf, W1f, Bf, W2f, Bhf, H)
        hb = _gru_cat(xb, hb, W1b, Bb, W2b, Bhb, H)
        ctx_ref[pl.ds(t, 1), :, 0:H] = hf[None]
        ctx_ref[pl.ds(T - 1 - t, 1), :, H:2 * H] = hb[None]
        return hf, hb

    jax.lax.fori_loop(0, T, step_bi, (h0, h0))

    W1o, Bo, W2o, Bho = w1o[:].astype(_BF), bo_[:], w2o[:].astype(_BF), bho_[:]

    hs_ref[pl.ds(0, 1)] = jnp.zeros((1, B, H), _F32)

    def step_o(k, h):
        c = ctx_ref[pl.ds(k, 1)][0]
        h = _gru_cat(c, h, W1o, Bo, W2o, Bho, H)
        hs_ref[pl.ds(k, 1)] = h[None]
        return h

    jax.lax.fori_loop(1, T, step_o, h0)


def _scans_tc(x, w1f, b_f, w2f, bh_f,
              w1b, b_b, w2b, bh_b,
              w1o, b_o, w2o, bh_o, interpret=False):
    T, B, H = x.shape
    body = functools.partial(_scans_body, T, B, H)
    return pl.pallas_call(
        body,
        out_shape=[
            jax.ShapeDtypeStruct((T, B, 2 * H), _F32),
            jax.ShapeDtypeStruct((T, B, H), _F32),
        ],
        compiler_params=pltpu.CompilerParams(
            vmem_limit_bytes=64 * 1024 * 1024),
        interpret=interpret,
    )(x, w1f, b_f, w2f, bh_f, w1b, b_b, w2b, bh_b, w1o, b_o, w2o, bh_o)


def _attn_body(T, Bc, H, Y,
               ctx_ref, hs_ref,
               wac, ba_, wah, way, wfc, wff, wfh, bfb,
               wy, byb, y_ref, ctxT_s, hsT_s, pc_s, q_s, s_s, wc_s):
    H2 = 2 * H
    ctxT_s[:] = jnp.transpose(ctx_ref[:], (1, 0, 2))
    hsT_s[:] = jnp.transpose(hs_ref[:], (1, 0, 2))
    ctx2 = ctx_ref[:].reshape(T * Bc, H2)
    pc_s[:] = (jnp.dot(ctx2, wac[:], preferred_element_type=_F32)
               + ba_[:]).reshape(T, Bc, H2)
    q_s[:] = jnp.dot(hs_ref[:].reshape(T * Bc, H), wah[:],
                     preferred_element_type=_F32).reshape(T, Bc, H2)
    wayb = way[:].reshape(1, 1, H2)

    s_s[:, 0:1, :] = jnp.zeros((Bc, 1, T), _F32)

    def step(j, _):
        qk = q_s[pl.ds(j - 1, 1)]
        sc = jnp.sum(jnp.tanh(pc_s[:] + qk) * wayb, axis=-1)
        s_s[:, pl.ds(j, 1), :] = sc.T[:, None, :]
        return 0

    jax.lax.fori_loop(1, T, step, 0, unroll=8)

    e = jnp.exp(s_s[:])
    s_s[:] = e / jnp.sum(e, axis=-1, keepdims=True)

    for b in range(Bc):
        wc_s[b] = jnp.dot(s_s[b], ctxT_s[b], preferred_element_type=_F32)

    M = T * Bc
    wc2 = wc_s[:].reshape(M, H2)
    h2 = hsT_s[:].reshape(M, H)
    lfc = jnp.dot(wc2, wfc[:], preferred_element_type=_F32)
    fw = jax.nn.sigmoid(
        jnp.dot(lfc, wff[:], preferred_element_type=_F32)
        + jnp.dot(h2, wfh[:], preferred_element_type=_F32) + bfb[:])
    hf = lfc * fw + h2
    y2 = jnp.dot(hf, wy[:], preferred_element_type=_F32) + byb[:]
    y_ref[:] = y2.reshape(Bc, T, Y)


def _attn_tc(context, hs, Wa_c, ba, Wa_h, Wa_y,
             Wf_c, Wf_f, Wf_h, bf, Wy, by, interpret=False):
    T, B, H2 = context.shape
    H = H2 // 2
    Y = Wy.shape[1]
    Bc = 8
    body = functools.partial(_attn_body, T, Bc, H, Y)
    full = lambda w: pl.BlockSpec(w.shape, lambda i: tuple(0 for _ in w.shape))
    return pl.pallas_call(
        body,
        grid=(B // Bc,),
        in_specs=[
            pl.BlockSpec((T, Bc, H2), lambda i: (0, i, 0)),
            pl.BlockSpec((T, Bc, H), lambda i: (0, i, 0)),
            full(Wa_c), full(ba), full(Wa_h), full(Wa_y),
            full(Wf_c), full(Wf_f), full(Wf_h), full(bf),
            full(Wy), full(by),
        ],
        out_specs=pl.BlockSpec((Bc, T, Y), lambda i: (i, 0, 0)),
        out_shape=jax.ShapeDtypeStruct((B, T, Y), _F32),
        scratch_shapes=[
            pltpu.VMEM((Bc, T, H2), _F32),
            pltpu.VMEM((Bc, T, H), _F32),
            pltpu.VMEM((T, Bc, H2), _F32),
            pltpu.VMEM((T, Bc, H2), _F32),
            pltpu.VMEM((Bc, T, T), _F32),
            pltpu.VMEM((Bc, T, H2), _F32),
        ],
        compiler_params=pltpu.CompilerParams(
            vmem_limit_bytes=64 * 1024 * 1024),
        interpret=interpret,
    )(context, hs, Wa_c, ba, Wa_h, Wa_y, Wf_c, Wf_f, Wf_h, bf, Wy, by)


def kernel(inputs, We, Wx_f, Wh_f, b_f, Wxh_f, Whh_f, bh_f,
           Wx_b, Wh_b, b_b, Wxh_b, Whh_b, bh_b,
           Wx_o, Wh_o, b_o, Wxh_o, Whh_o, bh_o, Wy, by,
           Wa_h, Wa_c, ba, Wa_y, Wf_h, Wf_c, Wf_f, bf):
    T, B = inputs.shape
    H = We.shape[1]

    x = _gather_sc(We, inputs.reshape(T * B)).reshape(T, B, H)

    context, hs = _scans_tc(
        x,
        jnp.concatenate([Wx_f, Wh_f], axis=0), b_f,
        jnp.concatenate([Wxh_f, Whh_f], axis=0), bh_f,
        jnp.concatenate([Wx_b, Wh_b], axis=0), b_b,
        jnp.concatenate([Wxh_b, Whh_b], axis=0), bh_b,
        jnp.concatenate([Wx_o, Wh_o], axis=0), b_o,
        jnp.concatenate([Wxh_o, Whh_o], axis=0), bh_o)

    y_full = _attn_tc(context, hs, Wa_c, ba, Wa_h,
                      Wa_y.reshape(1, 2 * H), Wf_c, Wf_f, Wf_h, bf, Wy, by)
    return jnp.transpose(y_full[:, 1:, :], (1, 0, 2))


# speedup vs baseline: 1.0391x; 1.0391x over previous
import functools

import jax
import jax.numpy as jnp
from jax.experimental import pallas as pl
from jax.experimental.pallas import tpu as pltpu
from jax.experimental.pallas import tpu_sc as plsc

_F32 = jnp.float32


def _gather_sc(We, idx_flat):
    N = idx_flat.shape[0]
    H = We.shape[1]
    W = 128
    mesh = plsc.VectorSubcoreMesh(core_axis_name="core", subcore_axis_name="subcore")
    idx2 = idx_flat.reshape(1, N)

    @functools.partial(
        pl.kernel,
        out_type=jax.ShapeDtypeStruct((N, H), _F32),
        mesh=mesh,
    )
    def gather_kernel(table_hbm, i_hbm, o_hbm):
        def body(i_vmem, o_vmem):
            pltpu.sync_copy(table_hbm.at[i_vmem.at[0]], o_vmem)

        pltpu.emit_pipeline(
            body,
            grid=(N // W,),
            in_specs=[pl.BlockSpec((1, W), lambda i: (0, i))],
            out_specs=[pl.BlockSpec((W, H), lambda i: (i, 0))],
            core_axis_name=("core", "subcore"),
            dimension_semantics=(pltpu.PARALLEL,),
        )(i_hbm, o_hbm)

    return gather_kernel(We, idx2)


def _gru_cat(x, h, W1, b, W2, bh, n):
    rz = jax.nn.sigmoid(
        jnp.dot(jnp.concatenate([x, h], axis=1), W1,
                preferred_element_type=_F32) + b)
    r = rz[:, :n]
    z = rz[:, n:]
    hc = jnp.tanh(
        jnp.dot(jnp.concatenate([x, h * r], axis=1), W2,
                preferred_element_type=_F32) + bh)
    return z * h + (1.0 - z) * hc


def _scans_body(T, B, H,
                x_ref, w1f, bf_, w2f, bhf_,
                w1b, bb_, w2b, bhb_,
                w1o, bo_, w2o, bho_,
                ctx_ref, hs_ref):
    h0 = jnp.zeros((B, H), _F32)

    W1f, Bf, W2f, Bhf = w1f[:], bf_[:], w2f[:], bhf_[:]
    W1b, Bb, W2b, Bhb = w1b[:], bb_[:], w2b[:], bhb_[:]

    def step_bi(t, carry):
        hf, hb = carry
        xf = x_ref[pl.ds(t, 1)][0]
        xb = x_ref[pl.ds(T - 1 - t, 1)][0]
        hf = _gru_cat(xf, hf, W1f, Bf, W2f, Bhf, H)
        hb = _gru_cat(xb, hb, W1b, Bb, W2b, Bhb, H)
        ctx_ref[pl.ds(t, 1), :, 0:H] = hf[None]
        ctx_ref[pl.ds(T - 1 - t, 1), :, H:2 * H] = hb[None]
        return hf, hb

    jax.lax.fori_loop(0, T, step_bi, (h0, h0), unroll=2)

    W1o, Bo, W2o, Bho = w1o[:], bo_[:], w2o[:], bho_[:]

    hs_ref[pl.ds(0, 1)] = jnp.zeros((1, B, H), _F32)

    def step_o(k, h):
        c = ctx_ref[pl.ds(k, 1)][0]
        h = _gru_cat(c, h, W1o, Bo, W2o, Bho, H)
        hs_ref[pl.ds(k, 1)] = h[None]
        return h

    jax.lax.fori_loop(1, T, step_o, h0, unroll=2)


def _scans_tc(x, w1f, b_f, w2f, bh_f,
              w1b, b_b, w2b, bh_b,
              w1o, b_o, w2o, bh_o, interpret=False):
    T, B, H = x.shape
    body = functools.partial(_scans_body, T, B, H)
    return pl.pallas_call(
        body,
        out_shape=[
            jax.ShapeDtypeStruct((T, B, 2 * H), _F32),
            jax.ShapeDtypeStruct((T, B, H), _F32),
        ],
        compiler_params=pltpu.CompilerParams(
            vmem_limit_bytes=64 * 1024 * 1024),
        interpret=interpret,
    )(x, w1f, b_f, w2f, bh_f, w1b, b_b, w2b, bh_b, w1o, b_o, w2o, bh_o)


def _attn_body(T, Bc, H, Y,
               ctx_ref, hs_ref,
               wac, ba_, wah, way, wfc, wff, wfh, bfb,
               wy, byb, y_ref, ctxT_s, hsT_s, pc_s, q_s, s_s, wc_s):
    H2 = 2 * H
    ctxT_s[:] = jnp.transpose(ctx_ref[:], (1, 0, 2))
    hsT_s[:] = jnp.transpose(hs_ref[:], (1, 0, 2))
    ctx2 = ctx_ref[:].reshape(T * Bc, H2)
    pc_s[:] = (jnp.dot(ctx2, wac[:], preferred_element_type=_F32)
               + ba_[:]).reshape(T, Bc, H2)
    q_s[:] = jnp.dot(hs_ref[:].reshape(T * Bc, H), wah[:],
                     preferred_element_type=_F32).reshape(T, Bc, H2)
    wayb = way[:].reshape(1, 1, H2)

    s_s[:, 0:1, :] = jnp.zeros((Bc, 1, T), _F32)

    def step(j, _):
        qk = q_s[pl.ds(j - 1, 1)]
        sc = jnp.sum(jnp.tanh(pc_s[:] + qk) * wayb, axis=-1)
        s_s[:, pl.ds(j, 1), :] = sc.T[:, None, :]
        return 0

    jax.lax.fori_loop(1, T, step, 0, unroll=8)

    e = jnp.exp(s_s[:])
    s_s[:] = e / jnp.sum(e, axis=-1, keepdims=True)

    for b in range(Bc):
        wc_s[b] = jnp.dot(s_s[b], ctxT_s[b], preferred_element_type=_F32)

    M = T * Bc
    wc2 = wc_s[:].reshape(M, H2)
    h2 = hsT_s[:].reshape(M, H)
    lfc = jnp.dot(wc2, wfc[:], preferred_element_type=_F32)
    fw = jax.nn.sigmoid(
        jnp.dot(lfc, wff[:], preferred_element_type=_F32)
        + jnp.dot(h2, wfh[:], preferred_element_type=_F32) + bfb[:])
    hf = lfc * fw + h2
    y2 = jnp.dot(hf, wy[:], preferred_element_type=_F32) + byb[:]
    y_ref[:] = y2.reshape(Bc, T, Y)


def _attn_tc(context, hs, Wa_c, ba, Wa_h, Wa_y,
             Wf_c, Wf_f, Wf_h, bf, Wy, by, interpret=False):
    T, B, H2 = context.shape
    H = H2 // 2
    Y = Wy.shape[1]
    Bc = 8
    body = functools.partial(_attn_body, T, Bc, H, Y)
    full = lambda w: pl.BlockSpec(w.shape, lambda i: tuple(0 for _ in w.shape))
    return pl.pallas_call(
        body,
        grid=(B // Bc,),
        in_specs=[
            pl.BlockSpec((T, Bc, H2), lambda i: (0, i, 0)),
            pl.BlockSpec((T, Bc, H), lambda i: (0, i, 0)),
            full(Wa_c), full(ba), full(Wa_h), full(Wa_y),
            full(Wf_c), full(Wf_f), full(Wf_h), full(bf),
            full(Wy), full(by),
        ],
        out_specs=pl.BlockSpec((Bc, T, Y), lambda i: (i, 0, 0)),
        out_shape=jax.ShapeDtypeStruct((B, T, Y), _F32),
        scratch_shapes=[
            pltpu.VMEM((Bc, T, H2), _F32),
            pltpu.VMEM((Bc, T, H), _F32),
            pltpu.VMEM((T, Bc, H2), _F32),
            pltpu.VMEM((T, Bc, H2), _F32),
            pltpu.VMEM((Bc, T, T), _F32),
            pltpu.VMEM((Bc, T, H2), _F32),
        ],
        compiler_params=pltpu.CompilerParams(
            vmem_limit_bytes=64 * 1024 * 1024),
        interpret=interpret,
    )(context, hs, Wa_c, ba, Wa_h, Wa_y, Wf_c, Wf_f, Wf_h, bf, Wy, by)


def kernel(inputs, We, Wx_f, Wh_f, b_f, Wxh_f, Whh_f, bh_f,
           Wx_b, Wh_b, b_b, Wxh_b, Whh_b, bh_b,
           Wx_o, Wh_o, b_o, Wxh_o, Whh_o, bh_o, Wy, by,
           Wa_h, Wa_c, ba, Wa_y, Wf_h, Wf_c, Wf_f, bf):
    T, B = inputs.shape
    H = We.shape[1]

    x = _gather_sc(We, inputs.reshape(T * B)).reshape(T, B, H)

    context, hs = _scans_tc(
        x,
        jnp.concatenate([Wx_f, Wh_f], axis=0), b_f,
        jnp.concatenate([Wxh_f, Whh_f], axis=0), bh_f,
        jnp.concatenate([Wx_b, Wh_b], axis=0), b_b,
        jnp.concatenate([Wxh_b, Whh_b], axis=0), bh_b,
        jnp.concatenate([Wx_o, Wh_o], axis=0), b_o,
        jnp.concatenate([Wxh_o, Whh_o], axis=0), bh_o)

    y_full = _attn_tc(context, hs, Wa_c, ba, Wa_h,
                      Wa_y.reshape(1, 2 * H), Wf_c, Wf_f, Wf_h, bf, Wy, by)
    return jnp.transpose(y_full[:, 1:, :], (1, 0, 2))


# speedup vs baseline: 1.0629x; 1.0229x over previous
import functools

import jax
import jax.numpy as jnp
from jax.experimental import pallas as pl
from jax.experimental.pallas import tpu as pltpu
from jax.experimental.pallas import tpu_sc as plsc

_F32 = jnp.float32


def _gather_sc(We, idx_flat):
    N = idx_flat.shape[0]
    H = We.shape[1]
    W = 128
    mesh = plsc.VectorSubcoreMesh(core_axis_name="core", subcore_axis_name="subcore")
    idx2 = idx_flat.reshape(1, N)

    @functools.partial(
        pl.kernel,
        out_type=jax.ShapeDtypeStruct((N, H), _F32),
        mesh=mesh,
    )
    def gather_kernel(table_hbm, i_hbm, o_hbm):
        def body(i_vmem, o_vmem):
            pltpu.sync_copy(table_hbm.at[i_vmem.at[0]], o_vmem)

        pltpu.emit_pipeline(
            body,
            grid=(N // W,),
            in_specs=[pl.BlockSpec((1, W), lambda i: (0, i))],
            out_specs=[pl.BlockSpec((W, H), lambda i: (i, 0))],
            core_axis_name=("core", "subcore"),
            dimension_semantics=(pltpu.PARALLEL,),
        )(i_hbm, o_hbm)

    return gather_kernel(We, idx2)


def _gru_cat(x, h, W1, b, W2, bh, n):
    rz = jax.nn.sigmoid(
        jnp.dot(jnp.concatenate([x, h], axis=1), W1,
                preferred_element_type=_F32) + b)
    r = rz[:, :n]
    z = rz[:, n:]
    hc = jnp.tanh(
        jnp.dot(jnp.concatenate([x, h * r], axis=1), W2,
                preferred_element_type=_F32) + bh)
    return z * h + (1.0 - z) * hc


def _scans_body(T, B, H,
                x_ref, w1f, bf_, w2f, bhf_,
                w1b, bb_, w2b, bhb_,
                w1o, bo_, w2o, bho_,
                ctx_ref, hs_ref):
    h0 = jnp.zeros((B, H), _F32)

    W1f, Bf, W2f, Bhf = w1f[:], bf_[:], w2f[:], bhf_[:]
    W1b, Bb, W2b, Bhb = w1b[:], bb_[:], w2b[:], bhb_[:]

    def step_bi(t, carry):
        hf, hb = carry
        xf = x_ref[pl.ds(t, 1)][0]
        xb = x_ref[pl.ds(T - 1 - t, 1)][0]
        hf = _gru_cat(xf, hf, W1f, Bf, W2f, Bhf, H)
        hb = _gru_cat(xb, hb, W1b, Bb, W2b, Bhb, H)
        ctx_ref[pl.ds(t, 1), :, 0:H] = hf[None]
        ctx_ref[pl.ds(T - 1 - t, 1), :, H:2 * H] = hb[None]
        return hf, hb

    jax.lax.fori_loop(0, T, step_bi, (h0, h0), unroll=4)

    W1o, Bo, W2o, Bho = w1o[:], bo_[:], w2o[:], bho_[:]

    hs_ref[pl.ds(0, 1)] = jnp.zeros((1, B, H), _F32)

    def step_o(k, h):
        c = ctx_ref[pl.ds(k, 1)][0]
        h = _gru_cat(c, h, W1o, Bo, W2o, Bho, H)
        hs_ref[pl.ds(k, 1)] = h[None]
        return h

    jax.lax.fori_loop(1, T, step_o, h0, unroll=4)


def _scans_tc(x, w1f, b_f, w2f, bh_f,
              w1b, b_b, w2b, bh_b,
              w1o, b_o, w2o, bh_o, interpret=False):
    T, B, H = x.shape
    body = functools.partial(_scans_body, T, B, H)
    return pl.pallas_call(
        body,
        out_shape=[
            jax.ShapeDtypeStruct((T, B, 2 * H), _F32),
            jax.ShapeDtypeStruct((T, B, H), _F32),
        ],
        compiler_params=pltpu.CompilerParams(
            vmem_limit_bytes=64 * 1024 * 1024),
        interpret=interpret,
    )(x, w1f, b_f, w2f, bh_f, w1b, b_b, w2b, bh_b, w1o, b_o, w2o, bh_o)


def _attn_body(T, Bc, H, Y,
               ctx_ref, hs_ref,
               wac, ba_, wah, way, wfc, wff, wfh, bfb,
               wy, byb, y_ref, ctxT_s, hsT_s, pc_s, q_s, s_s, wc_s):
    H2 = 2 * H
    ctxT_s[:] = jnp.transpose(ctx_ref[:], (1, 0, 2))
    hsT_s[:] = jnp.transpose(hs_ref[:], (1, 0, 2))
    ctx2 = ctx_ref[:].reshape(T * Bc, H2)
    pc_s[:] = (jnp.dot(ctx2, wac[:], preferred_element_type=_F32)
               + ba_[:]).reshape(T, Bc, H2)
    q_s[:] = jnp.dot(hs_ref[:].reshape(T * Bc, H), wah[:],
                     preferred_element_type=_F32).reshape(T, Bc, H2)
    wayb = way[:].reshape(1, 1, H2)

    s_s[:, 0:1, :] = jnp.zeros((Bc, 1, T), _F32)

    def step(j, _):
        qk = q_s[pl.ds(j - 1, 1)]
        sc = jnp.sum(jnp.tanh(pc_s[:] + qk) * wayb, axis=-1)
        s_s[:, pl.ds(j, 1), :] = sc.T[:, None, :]
        return 0

    jax.lax.fori_loop(1, T, step, 0, unroll=8)

    e = jnp.exp(s_s[:])
    s_s[:] = e / jnp.sum(e, axis=-1, keepdims=True)

    for b in range(Bc):
        wc_s[b] = jnp.dot(s_s[b], ctxT_s[b], preferred_element_type=_F32)

    M = T * Bc
    wc2 = wc_s[:].reshape(M, H2)
    h2 = hsT_s[:].reshape(M, H)
    lfc = jnp.dot(wc2, wfc[:], preferred_element_type=_F32)
    fw = jax.nn.sigmoid(
        jnp.dot(lfc, wff[:], preferred_element_type=_F32)
        + jnp.dot(h2, wfh[:], preferred_element_type=_F32) + bfb[:])
    hf = lfc * fw + h2
    y2 = jnp.dot(hf, wy[:], preferred_element_type=_F32) + byb[:]
    y_ref[:] = y2.reshape(Bc, T, Y)


def _attn_tc(context, hs, Wa_c, ba, Wa_h, Wa_y,
             Wf_c, Wf_f, Wf_h, bf, Wy, by, interpret=False):
    T, B, H2 = context.shape
    H = H2 // 2
    Y = Wy.shape[1]
    Bc = 8
    body = functools.partial(_attn_body, T, Bc, H, Y)
    full = lambda w: pl.BlockSpec(w.shape, lambda i: tuple(0 for _ in w.shape))
    return pl.pallas_call(
        body,
        grid=(B // Bc,),
        in_specs=[
            pl.BlockSpec((T, Bc, H2), lambda i: (0, i, 0)),
            pl.BlockSpec((T, Bc, H), lambda i: (0, i, 0)),
            full(Wa_c), full(ba), full(Wa_h), full(Wa_y),
            full(Wf_c), full(Wf_f), full(Wf_h), full(bf),
            full(Wy), full(by),
        ],
        out_specs=pl.BlockSpec((Bc, T, Y), lambda i: (i, 0, 0)),
        out_shape=jax.ShapeDtypeStruct((B, T, Y), _F32),
        scratch_shapes=[
            pltpu.VMEM((Bc, T, H2), _F32),
            pltpu.VMEM((Bc, T, H), _F32),
            pltpu.VMEM((T, Bc, H2), _F32),
            pltpu.VMEM((T, Bc, H2), _F32),
            pltpu.VMEM((Bc, T, T), _F32),
            pltpu.VMEM((Bc, T, H2), _F32),
        ],
        compiler_params=pltpu.CompilerParams(
            vmem_limit_bytes=64 * 1024 * 1024),
        interpret=interpret,
    )(context, hs, Wa_c, ba, Wa_h, Wa_y, Wf_c, Wf_f, Wf_h, bf, Wy, by)


def kernel(inputs, We, Wx_f, Wh_f, b_f, Wxh_f, Whh_f, bh_f,
           Wx_b, Wh_b, b_b, Wxh_b, Whh_b, bh_b,
           Wx_o, Wh_o, b_o, Wxh_o, Whh_o, bh_o, Wy, by,
           Wa_h, Wa_c, ba, Wa_y, Wf_h, Wf_c, Wf_f, bf):
    T, B = inputs.shape
    H = We.shape[1]

    x = _gather_sc(We, inputs.reshape(T * B)).reshape(T, B, H)

    context, hs = _scans_tc(
        x,
        jnp.concatenate([Wx_f, Wh_f], axis=0), b_f,
        jnp.concatenate([Wxh_f, Whh_f], axis=0), bh_f,
        jnp.concatenate([Wx_b, Wh_b], axis=0), b_b,
        jnp.concatenate([Wxh_b, Whh_b], axis=0), bh_b,
        jnp.concatenate([Wx_o, Wh_o], axis=0), b_o,
        jnp.concatenate([Wxh_o, Whh_o], axis=0), bh_o)

    y_full = _attn_tc(context, hs, Wa_c, ba, Wa_h,
                      Wa_y.reshape(1, 2 * H), Wf_c, Wf_f, Wf_h, bf, Wy, by)
    return jnp.transpose(y_full[:, 1:, :], (1, 0, 2))


# speedup vs baseline: 1.0812x; 1.0172x over previous
import functools

import jax
import jax.numpy as jnp
from jax.experimental import pallas as pl
from jax.experimental.pallas import tpu as pltpu
from jax.experimental.pallas import tpu_sc as plsc

_F32 = jnp.float32


def _gather_sc(We, idx_flat):
    N = idx_flat.shape[0]
    H = We.shape[1]
    W = 128
    mesh = plsc.VectorSubcoreMesh(core_axis_name="core", subcore_axis_name="subcore")
    idx2 = idx_flat.reshape(1, N)

    @functools.partial(
        pl.kernel,
        out_type=jax.ShapeDtypeStruct((N, H), _F32),
        mesh=mesh,
    )
    def gather_kernel(table_hbm, i_hbm, o_hbm):
        def body(i_vmem, o_vmem):
            pltpu.sync_copy(table_hbm.at[i_vmem.at[0]], o_vmem)

        pltpu.emit_pipeline(
            body,
            grid=(N // W,),
            in_specs=[pl.BlockSpec((1, W), lambda i: (0, i))],
            out_specs=[pl.BlockSpec((W, H), lambda i: (i, 0))],
            core_axis_name=("core", "subcore"),
            dimension_semantics=(pltpu.PARALLEL,),
        )(i_hbm, o_hbm)

    return gather_kernel(We, idx2)


def _gru_cat(x, h, W1, b, W2, bh, n):
    rz = jax.nn.sigmoid(
        jnp.dot(jnp.concatenate([x, h], axis=1), W1,
                preferred_element_type=_F32) + b)
    r = rz[:, :n]
    z = rz[:, n:]
    hc = jnp.tanh(
        jnp.dot(jnp.concatenate([x, h * r], axis=1), W2,
                preferred_element_type=_F32) + bh)
    return z * h + (1.0 - z) * hc


def _scans_body(T, B, H,
                x_ref, w1f, bf_, w2f, bhf_,
                w1b, bb_, w2b, bhb_,
                w1o, bo_, w2o, bho_,
                ctx_ref, hs_ref):
    h0 = jnp.zeros((B, H), _F32)

    W1f, Bf, W2f, Bhf = w1f[:], bf_[:], w2f[:], bhf_[:]
    W1b, Bb, W2b, Bhb = w1b[:], bb_[:], w2b[:], bhb_[:]

    def step_bi(t, carry):
        hf, hb = carry
        xf = x_ref[pl.ds(t, 1)][0]
        xb = x_ref[pl.ds(T - 1 - t, 1)][0]
        hf = _gru_cat(xf, hf, W1f, Bf, W2f, Bhf, H)
        hb = _gru_cat(xb, hb, W1b, Bb, W2b, Bhb, H)
        ctx_ref[pl.ds(t, 1), :, 0:H] = hf[None]
        ctx_ref[pl.ds(T - 1 - t, 1), :, H:2 * H] = hb[None]
        return hf, hb

    jax.lax.fori_loop(0, T, step_bi, (h0, h0), unroll=8)

    W1o, Bo, W2o, Bho = w1o[:], bo_[:], w2o[:], bho_[:]

    hs_ref[pl.ds(0, 1)] = jnp.zeros((1, B, H), _F32)

    def step_o(k, h):
        c = ctx_ref[pl.ds(k, 1)][0]
        h = _gru_cat(c, h, W1o, Bo, W2o, Bho, H)
        hs_ref[pl.ds(k, 1)] = h[None]
        return h

    jax.lax.fori_loop(1, T, step_o, h0, unroll=8)


def _scans_tc(x, w1f, b_f, w2f, bh_f,
              w1b, b_b, w2b, bh_b,
              w1o, b_o, w2o, bh_o, interpret=False):
    T, B, H = x.shape
    body = functools.partial(_scans_body, T, B, H)
    return pl.pallas_call(
        body,
        out_shape=[
            jax.ShapeDtypeStruct((T, B, 2 * H), _F32),
            jax.ShapeDtypeStruct((T, B, H), _F32),
        ],
        compiler_params=pltpu.CompilerParams(
            vmem_limit_bytes=64 * 1024 * 1024),
        interpret=interpret,
    )(x, w1f, b_f, w2f, bh_f, w1b, b_b, w2b, bh_b, w1o, b_o, w2o, bh_o)


def _attn_body(T, Bc, H, Y,
               ctx_ref, hs_ref,
               wac, ba_, wah, way, wfc, wff, wfh, bfb,
               wy, byb, y_ref, ctxT_s, hsT_s, pc_s, q_s, s_s, wc_s):
    H2 = 2 * H
    ctxT_s[:] = jnp.transpose(ctx_ref[:], (1, 0, 2))
    hsT_s[:] = jnp.transpose(hs_ref[:], (1, 0, 2))
    ctx2 = ctx_ref[:].reshape(T * Bc, H2)
    pc_s[:] = (jnp.dot(ctx2, wac[:], preferred_element_type=_F32)
               + ba_[:]).reshape(T, Bc, H2)
    q_s[:] = jnp.dot(hs_ref[:].reshape(T * Bc, H), wah[:],
                     preferred_element_type=_F32).reshape(T, Bc, H2)
    wayb = way[:].reshape(1, 1, H2)

    s_s[:, 0:1, :] = jnp.zeros((Bc, 1, T), _F32)

    def step(j, _):
        qk = q_s[pl.ds(j - 1, 1)]
        sc = jnp.sum(jnp.tanh(pc_s[:] + qk) * wayb, axis=-1)
        s_s[:, pl.ds(j, 1), :] = sc.T[:, None, :]
        return 0

    jax.lax.fori_loop(1, T, step, 0, unroll=8)

    e = jnp.exp(s_s[:])
    s_s[:] = e / jnp.sum(e, axis=-1, keepdims=True)

    for b in range(Bc):
        wc_s[b] = jnp.dot(s_s[b], ctxT_s[b], preferred_element_type=_F32)

    M = T * Bc
    wc2 = wc_s[:].reshape(M, H2)
    h2 = hsT_s[:].reshape(M, H)
    lfc = jnp.dot(wc2, wfc[:], preferred_element_type=_F32)
    fw = jax.nn.sigmoid(
        jnp.dot(lfc, wff[:], preferred_element_type=_F32)
        + jnp.dot(h2, wfh[:], preferred_element_type=_F32) + bfb[:])
    hf = lfc * fw + h2
    y2 = jnp.dot(hf, wy[:], preferred_element_type=_F32) + byb[:]
    y_ref[:] = y2.reshape(Bc, T, Y)


def _attn_tc(context, hs, Wa_c, ba, Wa_h, Wa_y,
             Wf_c, Wf_f, Wf_h, bf, Wy, by, interpret=False):
    T, B, H2 = context.shape
    H = H2 // 2
    Y = Wy.shape[1]
    Bc = 8
    body = functools.partial(_attn_body, T, Bc, H, Y)
    full = lambda w: pl.BlockSpec(w.shape, lambda i: tuple(0 for _ in w.shape))
    return pl.pallas_call(
        body,
        grid=(B // Bc,),
        in_specs=[
            pl.BlockSpec((T, Bc, H2), lambda i: (0, i, 0)),
            pl.BlockSpec((T, Bc, H), lambda i: (0, i, 0)),
            full(Wa_c), full(ba), full(Wa_h), full(Wa_y),
            full(Wf_c), full(Wf_f), full(Wf_h), full(bf),
            full(Wy), full(by),
        ],
        out_specs=pl.BlockSpec((Bc, T, Y), lambda i: (i, 0, 0)),
        out_shape=jax.ShapeDtypeStruct((B, T, Y), _F32),
        scratch_shapes=[
            pltpu.VMEM((Bc, T, H2), _F32),
            pltpu.VMEM((Bc, T, H), _F32),
            pltpu.VMEM((T, Bc, H2), _F32),
            pltpu.VMEM((T, Bc, H2), _F32),
            pltpu.VMEM((Bc, T, T), _F32),
            pltpu.VMEM((Bc, T, H2), _F32),
        ],
        compiler_params=pltpu.CompilerParams(
            vmem_limit_bytes=64 * 1024 * 1024),
        interpret=interpret,
    )(context, hs, Wa_c, ba, Wa_h, Wa_y, Wf_c, Wf_f, Wf_h, bf, Wy, by)


def kernel(inputs, We, Wx_f, Wh_f, b_f, Wxh_f, Whh_f, bh_f,
           Wx_b, Wh_b, b_b, Wxh_b, Whh_b, bh_b,
           Wx_o, Wh_o, b_o, Wxh_o, Whh_o, bh_o, Wy, by,
           Wa_h, Wa_c, ba, Wa_y, Wf_h, Wf_c, Wf_f, bf):
    T, B = inputs.shape
    H = We.shape[1]

    x = _gather_sc(We, inputs.reshape(T * B)).reshape(T, B, H)

    context, hs = _scans_tc(
        x,
        jnp.concatenate([Wx_f, Wh_f], axis=0), b_f,
        jnp.concatenate([Wxh_f, Whh_f], axis=0), bh_f,
        jnp.concatenate([Wx_b, Wh_b], axis=0), b_b,
        jnp.concatenate([Wxh_b, Whh_b], axis=0), bh_b,
        jnp.concatenate([Wx_o, Wh_o], axis=0), b_o,
        jnp.concatenate([Wxh_o, Whh_o], axis=0), bh_o)

    y_full = _attn_tc(context, hs, Wa_c, ba, Wa_h,
                      Wa_y.reshape(1, 2 * H), Wf_c, Wf_f, Wf_h, bf, Wy, by)
    return jnp.transpose(y_full[:, 1:, :], (1, 0, 2))


# speedup vs baseline: 1.0876x; 1.0059x over previous
import functools

import jax
import jax.numpy as jnp
from jax.experimental import pallas as pl
from jax.experimental.pallas import tpu as pltpu
from jax.experimental.pallas import tpu_sc as plsc

_F32 = jnp.float32


def _gather_sc(We, idx_flat):
    N = idx_flat.shape[0]
    H = We.shape[1]
    W = 128
    mesh = plsc.VectorSubcoreMesh(core_axis_name="core", subcore_axis_name="subcore")
    idx2 = idx_flat.reshape(1, N)

    @functools.partial(
        pl.kernel,
        out_type=jax.ShapeDtypeStruct((N, H), _F32),
        mesh=mesh,
    )
    def gather_kernel(table_hbm, i_hbm, o_hbm):
        def body(i_vmem, o_vmem):
            pltpu.sync_copy(table_hbm.at[i_vmem.at[0]], o_vmem)

        pltpu.emit_pipeline(
            body,
            grid=(N // W,),
            in_specs=[pl.BlockSpec((1, W), lambda i: (0, i))],
            out_specs=[pl.BlockSpec((W, H), lambda i: (i, 0))],
            core_axis_name=("core", "subcore"),
            dimension_semantics=(pltpu.PARALLEL,),
        )(i_hbm, o_hbm)

    return gather_kernel(We, idx2)


def _gru_cat(x, h, W1, b, W2, bh, n):
    rz = jax.nn.sigmoid(
        jnp.dot(jnp.concatenate([x, h], axis=1), W1,
                preferred_element_type=_F32) + b)
    r = rz[:, :n]
    z = rz[:, n:]
    hc = jnp.tanh(
        jnp.dot(jnp.concatenate([x, h * r], axis=1), W2,
                preferred_element_type=_F32) + bh)
    return z * h + (1.0 - z) * hc


def _scans_body(T, B, H,
                x_ref, w1f, bf_, w2f, bhf_,
                w1b, bb_, w2b, bhb_,
                w1o, bo_, w2o, bho_,
                ctx_ref, hs_ref):
    h0 = jnp.zeros((B, H), _F32)

    W1f, Bf, W2f, Bhf = w1f[:], bf_[:], w2f[:], bhf_[:]
    W1b, Bb, W2b, Bhb = w1b[:], bb_[:], w2b[:], bhb_[:]

    def step_bi(t, carry):
        hf, hb = carry
        xf = x_ref[pl.ds(t, 1)][0]
        xb = x_ref[pl.ds(T - 1 - t, 1)][0]
        hf = _gru_cat(xf, hf, W1f, Bf, W2f, Bhf, H)
        hb = _gru_cat(xb, hb, W1b, Bb, W2b, Bhb, H)
        ctx_ref[pl.ds(t, 1), :, 0:H] = hf[None]
        ctx_ref[pl.ds(T - 1 - t, 1), :, H:2 * H] = hb[None]
        return hf, hb

    jax.lax.fori_loop(0, T, step_bi, (h0, h0), unroll=16)

    W1o, Bo, W2o, Bho = w1o[:], bo_[:], w2o[:], bho_[:]

    hs_ref[pl.ds(0, 1)] = jnp.zeros((1, B, H), _F32)

    def step_o(k, h):
        c = ctx_ref[pl.ds(k, 1)][0]
        h = _gru_cat(c, h, W1o, Bo, W2o, Bho, H)
        hs_ref[pl.ds(k, 1)] = h[None]
        return h

    jax.lax.fori_loop(1, T, step_o, h0, unroll=16)


def _scans_tc(x, w1f, b_f, w2f, bh_f,
              w1b, b_b, w2b, bh_b,
              w1o, b_o, w2o, bh_o, interpret=False):
    T, B, H = x.shape
    body = functools.partial(_scans_body, T, B, H)
    return pl.pallas_call(
        body,
        out_shape=[
            jax.ShapeDtypeStruct((T, B, 2 * H), _F32),
            jax.ShapeDtypeStruct((T, B, H), _F32),
        ],
        compiler_params=pltpu.CompilerParams(
            vmem_limit_bytes=64 * 1024 * 1024),
        interpret=interpret,
    )(x, w1f, b_f, w2f, bh_f, w1b, b_b, w2b, bh_b, w1o, b_o, w2o, bh_o)


def _attn_body(T, Bc, H, Y,
               ctx_ref, hs_ref,
               wac, ba_, wah, way, wfc, wff, wfh, bfb,
               wy, byb, y_ref, ctxT_s, hsT_s, pc_s, q_s, s_s, wc_s):
    H2 = 2 * H
    ctxT_s[:] = jnp.transpose(ctx_ref[:], (1, 0, 2))
    hsT_s[:] = jnp.transpose(hs_ref[:], (1, 0, 2))
    ctx2 = ctx_ref[:].reshape(T * Bc, H2)
    pc_s[:] = (jnp.dot(ctx2, wac[:], preferred_element_type=_F32)
               + ba_[:]).reshape(T, Bc, H2)
    q_s[:] = jnp.dot(hs_ref[:].reshape(T * Bc, H), wah[:],
                     preferred_element_type=_F32).reshape(T, Bc, H2)
    wayb = way[:].reshape(1, 1, H2)

    s_s[:, 0:1, :] = jnp.zeros((Bc, 1, T), _F32)

    def step(j, _):
        qk = q_s[pl.ds(j - 1, 1)]
        sc = jnp.sum(jnp.tanh(pc_s[:] + qk) * wayb, axis=-1)
        s_s[:, pl.ds(j, 1), :] = sc.T[:, None, :]
        return 0

    jax.lax.fori_loop(1, T, step, 0, unroll=8)

    e = jnp.exp(s_s[:])
    s_s[:] = e / jnp.sum(e, axis=-1, keepdims=True)

    for b in range(Bc):
        wc_s[b] = jnp.dot(s_s[b], ctxT_s[b], preferred_element_type=_F32)

    M = T * Bc
    wc2 = wc_s[:].reshape(M, H2)
    h2 = hsT_s[:].reshape(M, H)
    lfc = jnp.dot(wc2, wfc[:], preferred_element_type=_F32)
    fw = jax.nn.sigmoid(
        jnp.dot(lfc, wff[:], preferred_element_type=_F32)
        + jnp.dot(h2, wfh[:], preferred_element_type=_F32) + bfb[:])
    hf = lfc * fw + h2
    y2 = jnp.dot(hf, wy[:], preferred_element_type=_F32) + byb[:]
    y_ref[:] = y2.reshape(Bc, T, Y)


def _attn_tc(context, hs, Wa_c, ba, Wa_h, Wa_y,
             Wf_c, Wf_f, Wf_h, bf, Wy, by, interpret=False):
    T, B, H2 = context.shape
    H = H2 // 2
    Y = Wy.shape[1]
    Bc = 8
    body = functools.partial(_attn_body, T, Bc, H, Y)
    full = lambda w: pl.BlockSpec(w.shape, lambda i: tuple(0 for _ in w.shape))
    return pl.pallas_call(
        body,
        grid=(B // Bc,),
        in_specs=[
            pl.BlockSpec((T, Bc, H2), lambda i: (0, i, 0)),
            pl.BlockSpec((T, Bc, H), lambda i: (0, i, 0)),
            full(Wa_c), full(ba), full(Wa_h), full(Wa_y),
            full(Wf_c), full(Wf_f), full(Wf_h), full(bf),
            full(Wy), full(by),
        ],
        out_specs=pl.BlockSpec((Bc, T, Y), lambda i: (i, 0, 0)),
        out_shape=jax.ShapeDtypeStruct((B, T, Y), _F32),
        scratch_shapes=[
            pltpu.VMEM((Bc, T, H2), _F32),
            pltpu.VMEM((Bc, T, H), _F32),
            pltpu.VMEM((T, Bc, H2), _F32),
            pltpu.VMEM((T, Bc, H2), _F32),
            pltpu.VMEM((Bc, T, T), _F32),
            pltpu.VMEM((Bc, T, H2), _F32),
        ],
        compiler_params=pltpu.CompilerParams(
            vmem_limit_bytes=64 * 1024 * 1024),
        interpret=interpret,
    )(context, hs, Wa_c, ba, Wa_h, Wa_y, Wf_c, Wf_f, Wf_h, bf, Wy, by)


def kernel(inputs, We, Wx_f, Wh_f, b_f, Wxh_f, Whh_f, bh_f,
           Wx_b, Wh_b, b_b, Wxh_b, Whh_b, bh_b,
           Wx_o, Wh_o, b_o, Wxh_o, Whh_o, bh_o, Wy, by,
           Wa_h, Wa_c, ba, Wa_y, Wf_h, Wf_c, Wf_f, bf):
    T, B = inputs.shape
    H = We.shape[1]

    x = _gather_sc(We, inputs.reshape(T * B)).reshape(T, B, H)

    context, hs = _scans_tc(
        x,
        jnp.concatenate([Wx_f, Wh_f], axis=0), b_f,
        jnp.concatenate([Wxh_f, Whh_f], axis=0), bh_f,
        jnp.concatenate([Wx_b, Wh_b], axis=0), b_b,
        jnp.concatenate([Wxh_b, Whh_b], axis=0), bh_b,
        jnp.concatenate([Wx_o, Wh_o], axis=0), b_o,
        jnp.concatenate([Wxh_o, Whh_o], axis=0), bh_o)

    y_full = _attn_tc(context, hs, Wa_c, ba, Wa_h,
                      Wa_y.reshape(1, 2 * H), Wf_c, Wf_f, Wf_h, bf, Wy, by)
    return jnp.transpose(y_full[:, 1:, :], (1, 0, 2))


# speedup vs baseline: 1.0947x; 1.0065x over previous
import functools

import jax
import jax.numpy as jnp
from jax.experimental import pallas as pl
from jax.experimental.pallas import tpu as pltpu
from jax.experimental.pallas import tpu_sc as plsc

_F32 = jnp.float32


def _gather_sc(We, idx_flat):
    N = idx_flat.shape[0]
    H = We.shape[1]
    W = 128
    mesh = plsc.VectorSubcoreMesh(core_axis_name="core", subcore_axis_name="subcore")
    idx2 = idx_flat.reshape(1, N)

    @functools.partial(
        pl.kernel,
        out_type=jax.ShapeDtypeStruct((N, H), _F32),
        mesh=mesh,
    )
    def gather_kernel(table_hbm, i_hbm, o_hbm):
        def body(i_vmem, o_vmem):
            pltpu.sync_copy(table_hbm.at[i_vmem.at[0]], o_vmem)

        pltpu.emit_pipeline(
            body,
            grid=(N // W,),
            in_specs=[pl.BlockSpec((1, W), lambda i: (0, i))],
            out_specs=[pl.BlockSpec((W, H), lambda i: (i, 0))],
            core_axis_name=("core", "subcore"),
            dimension_semantics=(pltpu.PARALLEL,),
        )(i_hbm, o_hbm)

    return gather_kernel(We, idx2)


def _gru_cat(x, h, W1, b, W2, bh, n):
    rz = jax.nn.sigmoid(
        jnp.dot(jnp.concatenate([x, h], axis=1), W1,
                preferred_element_type=_F32) + b)
    r = rz[:, :n]
    z = rz[:, n:]
    hc = jnp.tanh(
        jnp.dot(jnp.concatenate([x, h * r], axis=1), W2,
                preferred_element_type=_F32) + bh)
    return z * h + (1.0 - z) * hc


def _scans_body(T, B, H,
                x_ref, w1f, bf_, w2f, bhf_,
                w1b, bb_, w2b, bhb_,
                w1o, bo_, w2o, bho_,
                ctx_ref, hs_ref):
    h0 = jnp.zeros((B, H), _F32)

    W1f, Bf, W2f, Bhf = w1f[:], bf_[:], w2f[:], bhf_[:]
    W1b, Bb, W2b, Bhb = w1b[:], bb_[:], w2b[:], bhb_[:]

    def step_bi(t, carry):
        hf, hb = carry
        xf = x_ref[pl.ds(t, 1)][0]
        xb = x_ref[pl.ds(T - 1 - t, 1)][0]
        hf = _gru_cat(xf, hf, W1f, Bf, W2f, Bhf, H)
        hb = _gru_cat(xb, hb, W1b, Bb, W2b, Bhb, H)
        ctx_ref[pl.ds(t, 1), :, 0:H] = hf[None]
        ctx_ref[pl.ds(T - 1 - t, 1), :, H:2 * H] = hb[None]
        return hf, hb

    jax.lax.fori_loop(0, T, step_bi, (h0, h0), unroll=16)

    W1o, Bo, W2o, Bho = w1o[:], bo_[:], w2o[:], bho_[:]

    hs_ref[pl.ds(0, 1)] = jnp.zeros((1, B, H), _F32)

    def step_o(k, h):
        c = ctx_ref[pl.ds(k, 1)][0]
        h = _gru_cat(c, h, W1o, Bo, W2o, Bho, H)
        hs_ref[pl.ds(k, 1)] = h[None]
        return h

    jax.lax.fori_loop(1, T, step_o, h0, unroll=16)


def _scans_tc(x, w1f, b_f, w2f, bh_f,
              w1b, b_b, w2b, bh_b,
              w1o, b_o, w2o, bh_o, interpret=False):
    T, B, H = x.shape
    body = functools.partial(_scans_body, T, B, H)
    return pl.pallas_call(
        body,
        out_shape=[
            jax.ShapeDtypeStruct((T, B, 2 * H), _F32),
            jax.ShapeDtypeStruct((T, B, H), _F32),
        ],
        compiler_params=pltpu.CompilerParams(
            vmem_limit_bytes=64 * 1024 * 1024),
        interpret=interpret,
    )(x, w1f, b_f, w2f, bh_f, w1b, b_b, w2b, bh_b, w1o, b_o, w2o, bh_o)


def _attn_body(T, Bc, H, Y,
               ctx_ref, hs_ref,
               wac, ba_, wah, way, wfc, wff, wfh, bfb,
               wy, byb, y_ref, ctxT_s, hsT_s, pc_s, q_s, s_s, wc_s):
    H2 = 2 * H
    ctxT_s[:] = jnp.transpose(ctx_ref[:], (1, 0, 2))
    hsT_s[:] = jnp.transpose(hs_ref[:], (1, 0, 2))
    ctx2 = ctx_ref[:].reshape(T * Bc, H2)
    pc_s[:] = (jnp.dot(ctx2, wac[:], preferred_element_type=_F32)
               + ba_[:]).reshape(T, Bc, H2)
    q_s[:] = jnp.dot(hs_ref[:].reshape(T * Bc, H), wah[:],
                     preferred_element_type=_F32).reshape(T, Bc, H2)
    wayb = way[:].reshape(1, 1, H2)

    s_s[:, 0:1, :] = jnp.zeros((Bc, 1, T), _F32)

    def step(j, _):
        qk = q_s[pl.ds(j - 1, 1)]
        sc = jnp.sum(jnp.tanh(pc_s[:] + qk) * wayb, axis=-1)
        s_s[:, pl.ds(j, 1), :] = sc.T[:, None, :]
        return 0

    jax.lax.fori_loop(1, T, step, 0, unroll=16)

    e = jnp.exp(s_s[:])
    s_s[:] = e / jnp.sum(e, axis=-1, keepdims=True)

    for b in range(Bc):
        wc_s[b] = jnp.dot(s_s[b], ctxT_s[b], preferred_element_type=_F32)

    M = T * Bc
    wc2 = wc_s[:].reshape(M, H2)
    h2 = hsT_s[:].reshape(M, H)
    lfc = jnp.dot(wc2, wfc[:], preferred_element_type=_F32)
    fw = jax.nn.sigmoid(
        jnp.dot(lfc, wff[:], preferred_element_type=_F32)
        + jnp.dot(h2, wfh[:], preferred_element_type=_F32) + bfb[:])
    hf = lfc * fw + h2
    y2 = jnp.dot(hf, wy[:], preferred_element_type=_F32) + byb[:]
    y_ref[:] = y2.reshape(Bc, T, Y)


def _attn_tc(context, hs, Wa_c, ba, Wa_h, Wa_y,
             Wf_c, Wf_f, Wf_h, bf, Wy, by, interpret=False):
    T, B, H2 = context.shape
    H = H2 // 2
    Y = Wy.shape[1]
    Bc = 8
    body = functools.partial(_attn_body, T, Bc, H, Y)
    full = lambda w: pl.BlockSpec(w.shape, lambda i: tuple(0 for _ in w.shape))
    return pl.pallas_call(
        body,
        grid=(B // Bc,),
        in_specs=[
            pl.BlockSpec((T, Bc, H2), lambda i: (0, i, 0)),
            pl.BlockSpec((T, Bc, H), lambda i: (0, i, 0)),
            full(Wa_c), full(ba), full(Wa_h), full(Wa_y),
            full(Wf_c), full(Wf_f), full(Wf_h), full(bf),
            full(Wy), full(by),
        ],
        out_specs=pl.BlockSpec((Bc, T, Y), lambda i: (i, 0, 0)),
        out_shape=jax.ShapeDtypeStruct((B, T, Y), _F32),
        scratch_shapes=[
            pltpu.VMEM((Bc, T, H2), _F32),
            pltpu.VMEM((Bc, T, H), _F32),
            pltpu.VMEM((T, Bc, H2), _F32),
            pltpu.VMEM((T, Bc, H2), _F32),
            pltpu.VMEM((Bc, T, T), _F32),
            pltpu.VMEM((Bc, T, H2), _F32),
        ],
        compiler_params=pltpu.CompilerParams(
            vmem_limit_bytes=64 * 1024 * 1024),
        interpret=interpret,
    )(context, hs, Wa_c, ba, Wa_h, Wa_y, Wf_c, Wf_f, Wf_h, bf, Wy, by)


def kernel(inputs, We, Wx_f, Wh_f, b_f, Wxh_f, Whh_f, bh_f,
           Wx_b, Wh_b, b_b, Wxh_b, Whh_b, bh_b,
           Wx_o, Wh_o, b_o, Wxh_o, Whh_o, bh_o, Wy, by,
           Wa_h, Wa_c, ba, Wa_y, Wf_h, Wf_c, Wf_f, bf):
    T, B = inputs.shape
    H = We.shape[1]

    x = _gather_sc(We, inputs.reshape(T * B)).reshape(T, B, H)

    context, hs = _scans_tc(
        x,
        jnp.concatenate([Wx_f, Wh_f], axis=0), b_f,
        jnp.concatenate([Wxh_f, Whh_f], axis=0), bh_f,
        jnp.concatenate([Wx_b, Wh_b], axis=0), b_b,
        jnp.concatenate([Wxh_b, Whh_b], axis=0), bh_b,
        jnp.concatenate([Wx_o, Wh_o], axis=0), b_o,
        jnp.concatenate([Wxh_o, Whh_o], axis=0), bh_o)

    y_full = _attn_tc(context, hs, Wa_c, ba, Wa_h,
                      Wa_y.reshape(1, 2 * H), Wf_c, Wf_f, Wf_h, bf, Wy, by)
    return jnp.transpose(y_full[:, 1:, :], (1, 0, 2))
